# Initial kernel scaffold; baseline (speedup 1.0000x reference)
import jax
import jax.numpy as jnp
from jax.experimental import pallas as pl


def kernel(node_feat, edge_feat, edge_index, w_n2l_W, w_n2l_b, w_e2l_W, w_e2l_b, conv_W, conv_b, out_W, out_b):
    raise NotImplementedError("write your pallas kernel here")


# speedup vs baseline: 1.4820x; 1.4820x over previous
import functools

import jax
import jax.numpy as jnp
from jax import lax
from jax.experimental import pallas as pl
from jax.experimental.pallas import tpu as pltpu
from jax.experimental.pallas import tpu_sc as plsc

N = 10000
E = 160000
EP = 163840
EPH = EP // 2
NPAD = 10240
LAT = 128
NC = 2
NS = 16
NW = NC * NS
CH = 128
PT_G = EP // NW
NCH_G = PT_G // CH
PT_S = EP // NS
NCH_S = PT_S // CH
ZCH = NPAD // NS // CH
BLK = 2048
NBLK = EPH // BLK
BLKN = 1000
NBLKN = N // BLKN

_mesh = plsc.VectorSubcoreMesh(core_axis_name="c", subcore_axis_name="s")


@functools.partial(
    pl.kernel,
    out_type=jax.ShapeDtypeStruct((EP, LAT), jnp.float32),
    mesh=_mesh,
    scratch_types=[
        pltpu.VMEM((NCH_S, CH), jnp.int32),
        pltpu.VMEM((NCH_G, CH), jnp.int32),
        pltpu.VMEM((CH, LAT), jnp.float32),
        pltpu.VMEM_SHARED((NPAD, LAT), jnp.float32),
    ],
)
def _sc_scatter_gather(curw, dsti, srci, zeros, g_out,
                       dst_v, src_v, rows_v, table):
    c = lax.axis_index("c")
    s = lax.axis_index("s")
    wid = s * NC + c
    pltpu.sync_copy(zeros, rows_v)
    for k in range(ZCH):
        pltpu.sync_copy(rows_v, table.at[pl.ds(s * (ZCH * CH) + k * CH, CH)])
    plsc.subcore_barrier()
    pltpu.sync_copy(dsti.at[s], dst_v)

    def scat(j, carry):
        pltpu.sync_copy(curw.at[pl.ds(s * PT_S + j * CH, CH)], rows_v)
        pltpu.sync_copy(rows_v, table.at[dst_v.at[j]], add=True)
        return carry

    lax.fori_loop(0, NCH_S, scat, 0)
    plsc.subcore_barrier()
    pltpu.sync_copy(srci.at[wid], src_v)

    def gath(j, carry):
        pltpu.sync_copy(table.at[src_v.at[j]], rows_v)
        pltpu.sync_copy(rows_v, g_out.at[pl.ds(wid * PT_G + j * CH, CH)])
        return carry

    lax.fori_loop(0, NCH_G, gath, 0)


@functools.partial(
    pl.kernel,
    out_type=jax.ShapeDtypeStruct((EP, LAT), jnp.float32),
    mesh=_mesh,
    scratch_types=[
        pltpu.VMEM((NCH_G, CH), jnp.int32),
        pltpu.VMEM((CH, LAT), jnp.float32),
        pltpu.SemaphoreType.DMA,
    ],
)
def _sc_gather(table_hbm, srci, g_out, src_v, rows_v, sem):
    c = lax.axis_index("c")
    s = lax.axis_index("s")
    wid = s * NC + c
    pltpu.sync_copy(srci.at[wid], src_v)

    def gath(j, carry):
        pltpu.async_copy(table_hbm.at[src_v.at[j]], rows_v, sem).wait()
        pltpu.sync_copy(rows_v, g_out.at[pl.ds(wid * PT_G + j * CH, CH)])
        return carry

    lax.fori_loop(0, NCH_G, gath, 0)


def _tc_nlin_body(x_ref, w_ref, b_ref, o_ref):
    o_ref[...] = jnp.dot(x_ref[...], w_ref[...],
                         preferred_element_type=jnp.float32) + b_ref[...]


def _tc_pre_body(g_ref, ef_ref, wet_ref, be_ref, wct_ref, im_ref, cw_ref):
    g = g_ref[...]
    ef = ef_ref[...]
    wet = wet_ref[...]
    be = be_ref[...]
    e0 = jnp.dot(ef[:, :16], wet, preferred_element_type=jnp.float32) + be
    e1 = jnp.dot(ef[:, 16:], wet, preferred_element_type=jnp.float32) + be
    im0 = g[:, :LAT] + e0
    im1 = g[:, LAT:] + e1
    im_ref[...] = jnp.concatenate([im0, im1], axis=1)
    c0 = jnp.maximum(im0, 0.0)
    c1 = jnp.maximum(im1, 0.0)
    wct = wct_ref[...]
    cw_ref[...] = jnp.concatenate(
        [jnp.dot(c0, wct, preferred_element_type=jnp.float32),
         jnp.dot(c1, wct, preferred_element_type=jnp.float32)], axis=1)


def _tc_level_body(g_ref, cw_ref, im_ref, wct_ref, cb_ref, o_ref, *, last):
    g = g_ref[...]
    cw = cw_ref[...]
    im = im_ref[...]
    cb = cb_ref[...]
    h0 = jnp.maximum(g[:, :LAT] - cw[:, LAT:] + cb + im[:, :LAT], 0.0)
    h1 = jnp.maximum(g[:, LAT:] - cw[:, :LAT] + cb + im[:, LAT:], 0.0)
    if last:
        o_ref[...] = jnp.concatenate([h0, h1], axis=1)
    else:
        wct = wct_ref[...]
        o_ref[...] = jnp.concatenate(
            [jnp.dot(h0, wct, preferred_element_type=jnp.float32),
             jnp.dot(h1, wct, preferred_element_type=jnp.float32)], axis=1)


def _tc_out_body(t_ref, wot_ref, ob_ref, o_ref):
    i = pl.program_id(0)
    x = jnp.maximum(t_ref[...], 0.0)
    h = jnp.maximum(jnp.dot(x, wot_ref[...],
                            preferred_element_type=jnp.float32) + ob_ref[...],
                    0.0)
    s = jnp.sum(h, axis=0, keepdims=True)

    @pl.when(i == 0)
    def _():
        o_ref[...] = s

    @pl.when(i > 0)
    def _():
        o_ref[...] = o_ref[...] + s

    @pl.when(i == NBLKN - 1)
    def _():
        o_ref[...] = jnp.maximum(o_ref[...], 0.0)


def _full(shape):
    return pl.BlockSpec(shape, lambda i: tuple(0 for _ in shape))


def kernel(node_feat, edge_feat, edge_index, w_n2l_W, w_n2l_b, w_e2l_W,
           w_e2l_b, conv_W, conv_b, out_W, out_b):
    src = edge_index[0].astype(jnp.int32)
    dst = edge_index[1].astype(jnp.int32)
    src_p = jnp.concatenate([src, jnp.zeros((EP - E,), jnp.int32)])
    dst_p = jnp.concatenate([dst, jnp.full((EP - E,), N, jnp.int32)])
    srci = src_p.reshape(NW, NCH_G, CH)
    dsti = dst_p.reshape(NS, NCH_S, CH)
    zeros = jnp.zeros((CH, LAT), jnp.float32)
    ef_p = jnp.concatenate(
        [edge_feat, jnp.zeros((EP - E, 16), jnp.float32)]).reshape(EPH, 32)
    wnt = w_n2l_W.T
    wet = w_e2l_W.T
    wct = conv_W.T
    wot = out_W.T
    bn = w_n2l_b[None, :]
    be = w_e2l_b[None, :]
    cb = conv_b[None, :]
    ob = out_b[None, :]

    nlin = pl.pallas_call(
        _tc_nlin_body,
        grid=(NBLKN,),
        in_specs=[pl.BlockSpec((BLKN, 128), lambda i: (i, 0)),
                  _full((128, LAT)), _full((1, LAT))],
        out_specs=pl.BlockSpec((BLKN, LAT), lambda i: (i, 0)),
        out_shape=jax.ShapeDtypeStruct((N, LAT), jnp.float32),
    )(node_feat, wnt, bn)

    g0 = _sc_gather(nlin, srci)

    im, cw = pl.pallas_call(
        _tc_pre_body,
        grid=(NBLK,),
        in_specs=[pl.BlockSpec((BLK, 2 * LAT), lambda i: (i, 0)),
                  pl.BlockSpec((BLK, 32), lambda i: (i, 0)),
                  _full((16, LAT)), _full((1, LAT)), _full((LAT, LAT))],
        out_specs=[pl.BlockSpec((BLK, 2 * LAT), lambda i: (i, 0)),
                   pl.BlockSpec((BLK, 2 * LAT), lambda i: (i, 0))],
        out_shape=[jax.ShapeDtypeStruct((EPH, 2 * LAT), jnp.float32),
                   jax.ShapeDtypeStruct((EPH, 2 * LAT), jnp.float32)],
    )(g0.reshape(EPH, 2 * LAT), ef_p, wet, be, wct)

    for lv in range(3):
        g = _sc_scatter_gather(cw.reshape(EP, LAT), dsti, srci, zeros)
        cw = pl.pallas_call(
            functools.partial(_tc_level_body, last=(lv == 2)),
            grid=(NBLK,),
            in_specs=[pl.BlockSpec((BLK, 2 * LAT), lambda i: (i, 0)),
                      pl.BlockSpec((BLK, 2 * LAT), lambda i: (i, 0)),
                      pl.BlockSpec((BLK, 2 * LAT), lambda i: (i, 0)),
                      _full((LAT, LAT)), _full((1, LAT))],
            out_specs=pl.BlockSpec((BLK, 2 * LAT), lambda i: (i, 0)),
            out_shape=jax.ShapeDtypeStruct((EPH, 2 * LAT), jnp.float32),
        )(g.reshape(EPH, 2 * LAT), cw, im, wct, cb)

    ar = jnp.arange(NPAD, dtype=jnp.int32)
    srci_fin = jnp.concatenate(
        [ar, jnp.zeros((EP - NPAD,), jnp.int32)]).reshape(NW, NCH_G, CH)
    g_fin = _sc_scatter_gather(cw.reshape(EP, LAT), dsti, srci_fin, zeros)
    table = g_fin[:N]

    y = pl.pallas_call(
        _tc_out_body,
        grid=(NBLKN,),
        in_specs=[pl.BlockSpec((BLKN, LAT), lambda i: (i, 0)),
                  _full((LAT, 64)), _full((1, 64))],
        out_specs=_full((1, 64)),
        out_shape=jax.ShapeDtypeStruct((1, 64), jnp.float32),
    )(table, wot, ob)
    return y


# speedup vs baseline: 1.6770x; 1.1316x over previous
import functools

import jax
import jax.numpy as jnp
from jax import lax
from jax.experimental import pallas as pl
from jax.experimental.pallas import tpu as pltpu
from jax.experimental.pallas import tpu_sc as plsc

N = 10000
E = 160000
EP = 163840
EPH = EP // 2
NPAD = 10240
LAT = 128
NC = 2
NS = 16
NW = NC * NS
CH = 128
PT_G = EP // NW
NCH_G = PT_G // CH
PT_S = EP // NS
NCH_S = PT_S // CH
ZCH = NPAD // NS // CH
BLK = 2048
NBLK = EPH // BLK
BLKN = 1000
NBLKN = N // BLKN

_mesh = plsc.VectorSubcoreMesh(core_axis_name="c", subcore_axis_name="s")


@functools.partial(
    pl.kernel,
    out_type=jax.ShapeDtypeStruct((EP, LAT), jnp.float32),
    mesh=_mesh,
    scratch_types=[
        pltpu.VMEM((NCH_S, CH), jnp.int32),
        pltpu.VMEM((NCH_G, CH), jnp.int32),
        pltpu.VMEM((CH, LAT), jnp.float32),
        pltpu.VMEM((CH, LAT), jnp.float32),
        pltpu.VMEM_SHARED((NPAD, LAT), jnp.float32),
        pltpu.SemaphoreType.DMA,
        pltpu.SemaphoreType.DMA,
    ],
)
def _sc_scatter_gather(curw, dsti, srci, zeros, g_out,
                       dst_v, src_v, rows0, rows1, table, sem0, sem1):
    c = lax.axis_index("c")
    s = lax.axis_index("s")
    wid = s * NC + c
    bufs = (rows0, rows1)
    sems = (sem0, sem1)
    pltpu.sync_copy(zeros, rows0)
    for k in range(ZCH):
        pltpu.sync_copy(rows0, table.at[pl.ds(s * (ZCH * CH) + k * CH, CH)])
    pltpu.sync_copy(dsti.at[s], dst_v)
    pltpu.sync_copy(srci.at[wid], src_v)
    plsc.subcore_barrier()
    sbase = s * PT_S
    pltpu.async_copy(curw.at[pl.ds(sbase, CH)], rows0, sem0)

    def scat(j2, carry):
        for b in range(2):
            j = j2 * 2 + b
            buf, sem = bufs[b], sems[b]
            nbuf, nsem = bufs[1 - b], sems[1 - b]
            pltpu.make_async_copy(curw.at[pl.ds(sbase, CH)], buf, sem).wait()

            @pl.when(j < NCH_S - 1)
            def _():
                pltpu.async_copy(
                    curw.at[pl.ds(sbase + (j + 1) * CH, CH)], nbuf, nsem)

            pltpu.sync_copy(buf, table.at[dst_v.at[j]], add=True)
        return carry

    lax.fori_loop(0, NCH_S // 2, scat, 0)
    plsc.subcore_barrier()
    gbase = wid * PT_G

    def gath(j2, carry):
        for b in range(2):
            j = j2 * 2 + b
            buf, sem = bufs[b], sems[b]

            @pl.when(j2 > 0)
            def _():
                pltpu.make_async_copy(
                    buf, g_out.at[pl.ds(gbase, CH)], sem).wait()

            pltpu.sync_copy(table.at[src_v.at[j]], buf)
            pltpu.async_copy(buf, g_out.at[pl.ds(gbase + j * CH, CH)], sem)
        return carry

    lax.fori_loop(0, NCH_G // 2, gath, 0)
    for b in range(2):
        pltpu.make_async_copy(
            bufs[b], g_out.at[pl.ds(gbase, CH)], sems[b]).wait()


@functools.partial(
    pl.kernel,
    out_type=jax.ShapeDtypeStruct((EP, LAT), jnp.float32),
    mesh=_mesh,
    scratch_types=[
        pltpu.VMEM((NCH_G, CH), jnp.int32),
        pltpu.VMEM((CH, LAT), jnp.float32),
        pltpu.VMEM((CH, LAT), jnp.float32),
        pltpu.SemaphoreType.DMA,
        pltpu.SemaphoreType.DMA,
        pltpu.SemaphoreType.DMA,
        pltpu.SemaphoreType.DMA,
    ],
)
def _sc_gather(table_hbm, srci, g_out,
               src_v, rows0, rows1, gsem0, gsem1, wsem0, wsem1):
    c = lax.axis_index("c")
    s = lax.axis_index("s")
    wid = s * NC + c
    bufs = (rows0, rows1)
    gsems = (gsem0, gsem1)
    wsems = (wsem0, wsem1)
    pltpu.sync_copy(srci.at[wid], src_v)
    gbase = wid * PT_G
    pltpu.async_copy(table_hbm.at[src_v.at[0]], rows0, gsem0)

    def gath(j2, carry):
        for b in range(2):
            j = j2 * 2 + b
            buf, gsem, wsem = bufs[b], gsems[b], wsems[b]
            nbuf, ngsem, nwsem = bufs[1 - b], gsems[1 - b], wsems[1 - b]
            pltpu.make_async_copy(
                table_hbm.at[src_v.at[0]], buf, gsem).wait()
            pltpu.async_copy(buf, g_out.at[pl.ds(gbase + j * CH, CH)], wsem)

            @pl.when(j > 0)
            def _():
                pltpu.make_async_copy(
                    nbuf, g_out.at[pl.ds(gbase, CH)], nwsem).wait()

            @pl.when(j < NCH_G - 1)
            def _():
                pltpu.async_copy(
                    table_hbm.at[src_v.at[j + 1]], nbuf, ngsem)
        return carry

    lax.fori_loop(0, NCH_G // 2, gath, 0)
    pltpu.make_async_copy(rows1, g_out.at[pl.ds(gbase, CH)], wsem1).wait()


def _tc_nlin_body(x_ref, w_ref, b_ref, o_ref):
    o_ref[...] = jnp.dot(x_ref[...], w_ref[...],
                         preferred_element_type=jnp.float32) + b_ref[...]


def _tc_pre_body(g_ref, ef_ref, wet_ref, be_ref, wct_ref, im_ref, cw_ref):
    g = g_ref[...]
    ef = ef_ref[...]
    wet = wet_ref[...]
    be = be_ref[...]
    e0 = jnp.dot(ef[:, :16], wet, preferred_element_type=jnp.float32) + be
    e1 = jnp.dot(ef[:, 16:], wet, preferred_element_type=jnp.float32) + be
    im0 = g[:, :LAT] + e0
    im1 = g[:, LAT:] + e1
    im_ref[...] = jnp.concatenate([im0, im1], axis=1)
    c0 = jnp.maximum(im0, 0.0)
    c1 = jnp.maximum(im1, 0.0)
    wct = wct_ref[...]
    cw_ref[...] = jnp.concatenate(
        [jnp.dot(c0, wct, preferred_element_type=jnp.float32),
         jnp.dot(c1, wct, preferred_element_type=jnp.float32)], axis=1)


def _tc_level_body(g_ref, cw_ref, im_ref, wct_ref, cb_ref, o_ref, *, last):
    g = g_ref[...]
    cw = cw_ref[...]
    im = im_ref[...]
    cb = cb_ref[...]
    h0 = jnp.maximum(g[:, :LAT] - cw[:, LAT:] + cb + im[:, :LAT], 0.0)
    h1 = jnp.maximum(g[:, LAT:] - cw[:, :LAT] + cb + im[:, LAT:], 0.0)
    if last:
        o_ref[...] = jnp.concatenate([h0, h1], axis=1)
    else:
        wct = wct_ref[...]
        o_ref[...] = jnp.concatenate(
            [jnp.dot(h0, wct, preferred_element_type=jnp.float32),
             jnp.dot(h1, wct, preferred_element_type=jnp.float32)], axis=1)


def _tc_out_body(t_ref, wot_ref, ob_ref, o_ref):
    i = pl.program_id(0)
    x = jnp.maximum(t_ref[...], 0.0)
    h = jnp.maximum(jnp.dot(x, wot_ref[...],
                            preferred_element_type=jnp.float32) + ob_ref[...],
                    0.0)
    s = jnp.sum(h, axis=0, keepdims=True)

    @pl.when(i == 0)
    def _():
        o_ref[...] = s

    @pl.when(i > 0)
    def _():
        o_ref[...] = o_ref[...] + s

    @pl.when(i == NBLKN - 1)
    def _():
        o_ref[...] = jnp.maximum(o_ref[...], 0.0)


def _full(shape):
    return pl.BlockSpec(shape, lambda i: tuple(0 for _ in shape))


def kernel(node_feat, edge_feat, edge_index, w_n2l_W, w_n2l_b, w_e2l_W,
           w_e2l_b, conv_W, conv_b, out_W, out_b):
    src = edge_index[0].astype(jnp.int32)
    dst = edge_index[1].astype(jnp.int32)
    src_p = jnp.concatenate([src, jnp.zeros((EP - E,), jnp.int32)])
    dst_p = jnp.concatenate([dst, jnp.full((EP - E,), N, jnp.int32)])
    srci = src_p.reshape(NW, NCH_G, CH)
    dsti = dst_p.reshape(NS, NCH_S, CH)
    zeros = jnp.zeros((CH, LAT), jnp.float32)
    ef_p = jnp.concatenate(
        [edge_feat, jnp.zeros((EP - E, 16), jnp.float32)]).reshape(EPH, 32)
    wnt = w_n2l_W.T
    wet = w_e2l_W.T
    wct = conv_W.T
    wot = out_W.T
    bn = w_n2l_b[None, :]
    be = w_e2l_b[None, :]
    cb = conv_b[None, :]
    ob = out_b[None, :]

    nlin = pl.pallas_call(
        _tc_nlin_body,
        grid=(NBLKN,),
        in_specs=[pl.BlockSpec((BLKN, 128), lambda i: (i, 0)),
                  _full((128, LAT)), _full((1, LAT))],
        out_specs=pl.BlockSpec((BLKN, LAT), lambda i: (i, 0)),
        out_shape=jax.ShapeDtypeStruct((N, LAT), jnp.float32),
    )(node_feat, wnt, bn)

    g0 = _sc_gather(nlin, srci)

    im, cw = pl.pallas_call(
        _tc_pre_body,
        grid=(NBLK,),
        in_specs=[pl.BlockSpec((BLK, 2 * LAT), lambda i: (i, 0)),
                  pl.BlockSpec((BLK, 32), lambda i: (i, 0)),
                  _full((16, LAT)), _full((1, LAT)), _full((LAT, LAT))],
        out_specs=[pl.BlockSpec((BLK, 2 * LAT), lambda i: (i, 0)),
                   pl.BlockSpec((BLK, 2 * LAT), lambda i: (i, 0))],
        out_shape=[jax.ShapeDtypeStruct((EPH, 2 * LAT), jnp.float32),
                   jax.ShapeDtypeStruct((EPH, 2 * LAT), jnp.float32)],
    )(g0.reshape(EPH, 2 * LAT), ef_p, wet, be, wct)

    for lv in range(3):
        g = _sc_scatter_gather(cw.reshape(EP, LAT), dsti, srci, zeros)
        cw = pl.pallas_call(
            functools.partial(_tc_level_body, last=(lv == 2)),
            grid=(NBLK,),
            in_specs=[pl.BlockSpec((BLK, 2 * LAT), lambda i: (i, 0)),
                      pl.BlockSpec((BLK, 2 * LAT), lambda i: (i, 0)),
                      pl.BlockSpec((BLK, 2 * LAT), lambda i: (i, 0)),
                      _full((LAT, LAT)), _full((1, LAT))],
            out_specs=pl.BlockSpec((BLK, 2 * LAT), lambda i: (i, 0)),
            out_shape=jax.ShapeDtypeStruct((EPH, 2 * LAT), jnp.float32),
        )(g.reshape(EPH, 2 * LAT), cw, im, wct, cb)

    ar = jnp.arange(NPAD, dtype=jnp.int32)
    srci_fin = jnp.concatenate(
        [ar, jnp.zeros((EP - NPAD,), jnp.int32)]).reshape(NW, NCH_G, CH)
    g_fin = _sc_scatter_gather(cw.reshape(EP, LAT), dsti, srci_fin, zeros)
    table = g_fin[:N]

    y = pl.pallas_call(
        _tc_out_body,
        grid=(NBLKN,),
        in_specs=[pl.BlockSpec((BLKN, LAT), lambda i: (i, 0)),
                  _full((LAT, 64)), _full((1, 64))],
        out_specs=_full((1, 64)),
        out_shape=jax.ShapeDtypeStruct((1, 64), jnp.float32),
    )(table, wot, ob)
    return y


# speedup vs baseline: 2.7372x; 1.6322x over previous
import functools

import jax
import jax.numpy as jnp
from jax import lax
from jax.experimental import pallas as pl
from jax.experimental.pallas import tpu as pltpu
from jax.experimental.pallas import tpu_sc as plsc

N = 10000
E = 160000
EP = 163840
EP2 = EP // 2
NPAD = 10240
LAT = 128
NC = 2
NS = 16
NSH = NS // 2
NWH = 16
CH = 128
PT_G = EP2 // NWH
NCH_G = PT_G // CH
PT_S = EP2 // NSH
NCH_S = PT_S // CH
ZCH = NPAD // NS // CH
LROWS = NPAD // NS
BLK = 2048
NBLK = EP2 // BLK
BLKN = 1024
NBLKN = NPAD // BLKN
BLKO = 1000
NBLKO = N // BLKO

_mesh = plsc.VectorSubcoreMesh(core_axis_name="c", subcore_axis_name="s")


def _zero_table(zeros, rows0, table, s):
    pltpu.sync_copy(zeros, rows0)
    for k in range(ZCH):
        pltpu.sync_copy(rows0, table.at[pl.ds(s * (ZCH * CH) + k * CH, CH)])


def _scatter_loop(vals, dst_v, table, bufs, sems, sidx):
    base = sidx * PT_S
    pltpu.async_copy(vals.at[pl.ds(base, CH)], bufs[0], sems[0])

    def scat(j2, carry):
        for b in range(2):
            j = j2 * 2 + b
            buf, sem = bufs[b], sems[b]
            nbuf, nsem = bufs[1 - b], sems[1 - b]
            pltpu.make_async_copy(vals.at[pl.ds(base, CH)], buf, sem).wait()

            @pl.when(j < NCH_S - 1)
            def _():
                pltpu.async_copy(
                    vals.at[pl.ds(base + (j + 1) * CH, CH)], nbuf, nsem)

            pltpu.sync_copy(buf, table.at[dst_v.at[j]], add=True)
        return carry

    lax.fori_loop(0, NCH_S // 2, scat, 0)


def _gather_loop(g_out, src_v, table, bufs, sems, widx):
    gbase = widx * PT_G

    def gath(j2, carry):
        for b in range(2):
            j = j2 * 2 + b
            buf, sem = bufs[b], sems[b]

            @pl.when(j2 > 0)
            def _():
                pltpu.make_async_copy(
                    buf, g_out.at[pl.ds(gbase, CH)], sem).wait()

            pltpu.sync_copy(table.at[src_v.at[j]], buf)
            pltpu.async_copy(buf, g_out.at[pl.ds(gbase + j * CH, CH)], sem)
        return carry

    lax.fori_loop(0, NCH_G // 2, gath, 0)
    for b in range(2):
        pltpu.make_async_copy(
            bufs[b], g_out.at[pl.ds(gbase, CH)], sems[b]).wait()


@functools.partial(
    pl.kernel,
    out_type=[jax.ShapeDtypeStruct((EP2, LAT), jnp.float32),
              jax.ShapeDtypeStruct((EP2, LAT), jnp.float32)],
    mesh=_mesh,
    scratch_types=[
        pltpu.VMEM((NCH_S, CH), jnp.int32),
        pltpu.VMEM((NCH_G, CH), jnp.int32),
        pltpu.VMEM((CH, LAT), jnp.float32),
        pltpu.VMEM((CH, LAT), jnp.float32),
        pltpu.VMEM_SHARED((NPAD, LAT), jnp.float32),
        pltpu.SemaphoreType.DMA,
        pltpu.SemaphoreType.DMA,
    ],
)
def _sc_scatter_gather(cw_e, cw_o, dsti_e, dsti_o, srci_e, srci_o, zeros,
                       g_e, g_o, dst_v, src_v, rows0, rows1, table,
                       sem0, sem1):
    c = lax.axis_index("c")
    s = lax.axis_index("s")
    bufs = (rows0, rows1)
    sems = (sem0, sem1)
    par = s // NSH
    s_h = s - par * NSH
    wid_h = s_h * NC + c
    _zero_table(zeros, rows0, table, s)

    @pl.when(par == 0)
    def _():
        pltpu.sync_copy(dsti_e.at[s_h], dst_v)
        pltpu.sync_copy(srci_e.at[wid_h], src_v)

    @pl.when(par == 1)
    def _():
        pltpu.sync_copy(dsti_o.at[s_h], dst_v)
        pltpu.sync_copy(srci_o.at[wid_h], src_v)

    plsc.subcore_barrier()

    @pl.when(par == 0)
    def _():
        _scatter_loop(cw_e, dst_v, table, bufs, sems, s_h)

    @pl.when(par == 1)
    def _():
        _scatter_loop(cw_o, dst_v, table, bufs, sems, s_h)

    plsc.subcore_barrier()

    @pl.when(par == 0)
    def _():
        _gather_loop(g_e, src_v, table, bufs, sems, wid_h)

    @pl.when(par == 1)
    def _():
        _gather_loop(g_o, src_v, table, bufs, sems, wid_h)


@functools.partial(
    pl.kernel,
    out_type=[jax.ShapeDtypeStruct((EP2, LAT), jnp.float32),
              jax.ShapeDtypeStruct((EP2, LAT), jnp.float32)],
    mesh=_mesh,
    scratch_types=[
        pltpu.VMEM((NCH_G, CH), jnp.int32),
        pltpu.VMEM((CH, LAT), jnp.float32),
        pltpu.VMEM((CH, LAT), jnp.float32),
        pltpu.VMEM_SHARED((NPAD, LAT), jnp.float32),
        pltpu.SemaphoreType.DMA,
        pltpu.SemaphoreType.DMA,
    ],
)
def _sc_load_gather(nlinp, srci_e, srci_o, g_e, g_o,
                    src_v, rows0, rows1, table, sem0, sem1):
    c = lax.axis_index("c")
    s = lax.axis_index("s")
    bufs = (rows0, rows1)
    sems = (sem0, sem1)
    par = s // NSH
    s_h = s - par * NSH
    wid_h = s_h * NC + c
    pltpu.sync_copy(nlinp.at[pl.ds(s * LROWS, LROWS)],
                    table.at[pl.ds(s * LROWS, LROWS)])

    @pl.when(par == 0)
    def _():
        pltpu.sync_copy(srci_e.at[wid_h], src_v)

    @pl.when(par == 1)
    def _():
        pltpu.sync_copy(srci_o.at[wid_h], src_v)

    plsc.subcore_barrier()

    @pl.when(par == 0)
    def _():
        _gather_loop(g_e, src_v, table, bufs, sems, wid_h)

    @pl.when(par == 1)
    def _():
        _gather_loop(g_o, src_v, table, bufs, sems, wid_h)


def _tc_nlin_body(x_ref, w_ref, b_ref, o_ref):
    o_ref[...] = jnp.dot(x_ref[...], w_ref[...],
                         preferred_element_type=jnp.float32) + b_ref[...]


def _tc_pre_body(ge_ref, go_ref, ef_ref, wet_ref, be_ref, wct_ref,
                 ime_ref, imo_ref, cwe_ref, cwo_ref):
    ef = ef_ref[...]
    wet = wet_ref[...]
    be = be_ref[...]
    wct = wct_ref[...]
    ime = ge_ref[...] + jnp.dot(ef[:, :16], wet,
                                preferred_element_type=jnp.float32) + be
    imo = go_ref[...] + jnp.dot(ef[:, 16:], wet,
                                preferred_element_type=jnp.float32) + be
    ime_ref[...] = ime
    imo_ref[...] = imo
    cwe_ref[...] = jnp.dot(jnp.maximum(ime, 0.0), wct,
                           preferred_element_type=jnp.float32)
    cwo_ref[...] = jnp.dot(jnp.maximum(imo, 0.0), wct,
                           preferred_element_type=jnp.float32)


def _tc_level_body(ge_ref, go_ref, cwe_ref, cwo_ref, ime_ref, imo_ref,
                   wct_ref, cb_ref, oe_ref, oo_ref, *, last):
    cb = cb_ref[...]
    he = jnp.maximum(ge_ref[...] - cwo_ref[...] + cb + ime_ref[...], 0.0)
    ho = jnp.maximum(go_ref[...] - cwe_ref[...] + cb + imo_ref[...], 0.0)
    if last:
        oe_ref[...] = he
        oo_ref[...] = ho
    else:
        wct = wct_ref[...]
        oe_ref[...] = jnp.dot(he, wct, preferred_element_type=jnp.float32)
        oo_ref[...] = jnp.dot(ho, wct, preferred_element_type=jnp.float32)


def _tc_out_body(t_ref, wot_ref, ob_ref, o_ref):
    i = pl.program_id(0)
    x = jnp.maximum(t_ref[...], 0.0)
    h = jnp.maximum(jnp.dot(x, wot_ref[...],
                            preferred_element_type=jnp.float32) + ob_ref[...],
                    0.0)
    s = jnp.sum(h, axis=0, keepdims=True)

    @pl.when(i == 0)
    def _():
        o_ref[...] = s

    @pl.when(i > 0)
    def _():
        o_ref[...] = o_ref[...] + s

    @pl.when(i == NBLKO - 1)
    def _():
        o_ref[...] = jnp.maximum(o_ref[...], 0.0)


def _full(shape):
    return pl.BlockSpec(shape, lambda i: tuple(0 for _ in shape))


def _eblk(width=LAT):
    return pl.BlockSpec((BLK, width), lambda i: (i, 0))


def kernel(node_feat, edge_feat, edge_index, w_n2l_W, w_n2l_b, w_e2l_W,
           w_e2l_b, conv_W, conv_b, out_W, out_b):
    src = edge_index[0].astype(jnp.int32)
    dst = edge_index[1].astype(jnp.int32)
    pad_e = EP2 - E // 2
    src_e = jnp.concatenate([src[0::2], jnp.zeros((pad_e,), jnp.int32)])
    src_o = jnp.concatenate([src[1::2], jnp.zeros((pad_e,), jnp.int32)])
    dst_e = jnp.concatenate([dst[0::2], jnp.full((pad_e,), N, jnp.int32)])
    dst_o = jnp.concatenate([dst[1::2], jnp.full((pad_e,), N, jnp.int32)])
    srci_e = src_e.reshape(NWH, NCH_G, CH)
    srci_o = src_o.reshape(NWH, NCH_G, CH)
    dsti_e = dst_e.reshape(NSH, NCH_S, CH)
    dsti_o = dst_o.reshape(NSH, NCH_S, CH)
    zeros = jnp.zeros((CH, LAT), jnp.float32)
    ef2 = jnp.concatenate(
        [edge_feat, jnp.zeros((EP - E, 16), jnp.float32)]).reshape(EP2, 32)
    nf_p = jnp.concatenate(
        [node_feat, jnp.zeros((NPAD - N, 128), jnp.float32)])
    wnt = w_n2l_W.T
    wet = w_e2l_W.T
    wct = conv_W.T
    wot = out_W.T
    bn = w_n2l_b[None, :]
    be = w_e2l_b[None, :]
    cb = conv_b[None, :]
    ob = out_b[None, :]

    nlinp = pl.pallas_call(
        _tc_nlin_body,
        grid=(NBLKN,),
        in_specs=[pl.BlockSpec((BLKN, 128), lambda i: (i, 0)),
                  _full((128, LAT)), _full((1, LAT))],
        out_specs=pl.BlockSpec((BLKN, LAT), lambda i: (i, 0)),
        out_shape=jax.ShapeDtypeStruct((NPAD, LAT), jnp.float32),
    )(nf_p, wnt, bn)

    g0_e, g0_o = _sc_load_gather(nlinp, srci_e, srci_o)

    im_e, im_o, cw_e, cw_o = pl.pallas_call(
        _tc_pre_body,
        grid=(NBLK,),
        in_specs=[_eblk(), _eblk(), _eblk(32),
                  _full((16, LAT)), _full((1, LAT)), _full((LAT, LAT))],
        out_specs=[_eblk(), _eblk(), _eblk(), _eblk()],
        out_shape=[jax.ShapeDtypeStruct((EP2, LAT), jnp.float32)] * 4,
    )(g0_e, g0_o, ef2, wet, be, wct)

    for lv in range(3):
        g_e, g_o = _sc_scatter_gather(cw_e, cw_o, dsti_e, dsti_o,
                                      srci_e, srci_o, zeros)
        cw_e, cw_o = pl.pallas_call(
            functools.partial(_tc_level_body, last=(lv == 2)),
            grid=(NBLK,),
            in_specs=[_eblk(), _eblk(), _eblk(), _eblk(), _eblk(), _eblk(),
                      _full((LAT, LAT)), _full((1, LAT))],
            out_specs=[_eblk(), _eblk()],
            out_shape=[jax.ShapeDtypeStruct((EP2, LAT), jnp.float32)] * 2,
        )(g_e, g_o, cw_e, cw_o, im_e, im_o, wct, cb)

    ar = jnp.arange(NPAD, dtype=jnp.int32)
    srci_fin_e = jnp.concatenate(
        [ar, jnp.zeros((NWH * PT_G - NPAD,), jnp.int32)]).reshape(
            NWH, NCH_G, CH)
    srci_fin_o = jnp.zeros((NWH, NCH_G, CH), jnp.int32)
    gf_e, _ = _sc_scatter_gather(cw_e, cw_o, dsti_e, dsti_o,
                                 srci_fin_e, srci_fin_o, zeros)
    table = gf_e[:N]

    y = pl.pallas_call(
        _tc_out_body,
        grid=(NBLKO,),
        in_specs=[pl.BlockSpec((BLKO, LAT), lambda i: (i, 0)),
                  _full((LAT, 64)), _full((1, 64))],
        out_specs=_full((1, 64)),
        out_shape=jax.ShapeDtypeStruct((1, 64), jnp.float32),
    )(table, wot, ob)
    return y


# speedup vs baseline: 3.2283x; 1.1794x over previous
import functools

import jax
import jax.numpy as jnp
from jax import lax
from jax.experimental import pallas as pl
from jax.experimental.pallas import tpu as pltpu
from jax.experimental.pallas import tpu_sc as plsc

N = 10000
E = 160000
EP = 163840
EP2 = EP // 2
NPAD = 10240
LAT = 128
NC = 2
NS = 16
NSH = NS // 2
NWH = 16
CH = 128
PT_G = EP2 // NWH
NCH_G = PT_G // CH
PT_S2 = EP2 // NS
NCH_S2 = PT_S2 // CH
ZCH = NPAD // NS // CH
LROWS = NPAD // NS
BLK = 2048
NBLK = EP2 // BLK
BLKN = 1024
NBLKN = NPAD // BLKN
BLKO = 1000
NBLKO = N // BLKO

_mesh = plsc.VectorSubcoreMesh(core_axis_name="c", subcore_axis_name="s")


def _zero_table(zeros, rows0, table, s):
    pltpu.sync_copy(zeros, rows0)
    for k in range(ZCH):
        pltpu.sync_copy(rows0, table.at[pl.ds(s * (ZCH * CH) + k * CH, CH)])


def _scatter_loop(vals, dst_v, table, bufs, sems, base, nch):
    pltpu.async_copy(vals.at[pl.ds(base, CH)], bufs[0], sems[0])

    def scat(j2, carry):
        for b in range(2):
            j = j2 * 2 + b
            buf, sem = bufs[b], sems[b]
            nbuf, nsem = bufs[1 - b], sems[1 - b]
            pltpu.make_async_copy(vals.at[pl.ds(base, CH)], buf, sem).wait()

            @pl.when(j < nch - 1)
            def _():
                pltpu.async_copy(
                    vals.at[pl.ds(base + (j + 1) * CH, CH)], nbuf, nsem)

            pltpu.sync_copy(buf, table.at[dst_v.at[j]], add=True)
        return carry

    lax.fori_loop(0, nch // 2, scat, 0)


def _gather_loop(g_out, src_v, table, bufs, sems, widx):
    gbase = widx * PT_G

    def gath(j2, carry):
        for b in range(2):
            j = j2 * 2 + b
            buf, sem = bufs[b], sems[b]

            @pl.when(j2 > 0)
            def _():
                pltpu.make_async_copy(
                    buf, g_out.at[pl.ds(gbase, CH)], sem).wait()

            pltpu.sync_copy(table.at[src_v.at[j]], buf)
            pltpu.async_copy(buf, g_out.at[pl.ds(gbase + j * CH, CH)], sem)
        return carry

    lax.fori_loop(0, NCH_G // 2, gath, 0)
    for b in range(2):
        pltpu.make_async_copy(
            bufs[b], g_out.at[pl.ds(gbase, CH)], sems[b]).wait()


@functools.partial(
    pl.kernel,
    out_type=[jax.ShapeDtypeStruct((EP2, LAT), jnp.float32),
              jax.ShapeDtypeStruct((EP2, LAT), jnp.float32)],
    mesh=_mesh,
    scratch_types=[
        pltpu.VMEM((NCH_S2, CH), jnp.int32),
        pltpu.VMEM((CH, LAT), jnp.float32),
        pltpu.VMEM((CH, LAT), jnp.float32),
        pltpu.VMEM_SHARED((NPAD, LAT), jnp.float32),
        pltpu.SemaphoreType.DMA,
        pltpu.SemaphoreType.DMA,
    ],
)
def _sc_scatter_half(cw_e, cw_o, dsti_e, dsti_o, zeros, part0, part1,
                     dst_v, rows0, rows1, table, sem0, sem1):
    c = lax.axis_index("c")
    s = lax.axis_index("s")
    bufs = (rows0, rows1)
    sems = (sem0, sem1)
    par = s // NSH
    s_h = s - par * NSH
    q = c * NSH + s_h
    _zero_table(zeros, rows0, table, s)

    @pl.when(par == 0)
    def _():
        pltpu.sync_copy(dsti_e.at[q], dst_v)

    @pl.when(par == 1)
    def _():
        pltpu.sync_copy(dsti_o.at[q], dst_v)

    plsc.subcore_barrier()

    @pl.when(par == 0)
    def _():
        _scatter_loop(cw_e, dst_v, table, bufs, sems, q * PT_S2, NCH_S2)

    @pl.when(par == 1)
    def _():
        _scatter_loop(cw_o, dst_v, table, bufs, sems, q * PT_S2, NCH_S2)

    plsc.subcore_barrier()

    @pl.when(c == 0)
    def _():
        pltpu.sync_copy(table.at[pl.ds(s * LROWS, LROWS)],
                        part0.at[pl.ds(s * LROWS, LROWS)])

    @pl.when(c == 1)
    def _():
        pltpu.sync_copy(table.at[pl.ds(s * LROWS, LROWS)],
                        part1.at[pl.ds(s * LROWS, LROWS)])


@functools.partial(
    pl.kernel,
    out_type=[jax.ShapeDtypeStruct((EP2, LAT), jnp.float32),
              jax.ShapeDtypeStruct((EP2, LAT), jnp.float32)],
    mesh=_mesh,
    scratch_types=[
        pltpu.VMEM((NCH_G, CH), jnp.int32),
        pltpu.VMEM((ZCH, CH), jnp.int32),
        pltpu.VMEM((CH, LAT), jnp.float32),
        pltpu.VMEM((CH, LAT), jnp.float32),
        pltpu.VMEM_SHARED((NPAD, LAT), jnp.float32),
        pltpu.SemaphoreType.DMA,
        pltpu.SemaphoreType.DMA,
    ],
)
def _sc_combine_gather(part0, part1, iota_idx, srci_e, srci_o, g_e, g_o,
                       src_v, iota_v, rows0, rows1, table, sem0, sem1):
    c = lax.axis_index("c")
    s = lax.axis_index("s")
    bufs = (rows0, rows1)
    sems = (sem0, sem1)
    par = s // NSH
    s_h = s - par * NSH
    wid_h = s_h * NC + c
    pltpu.sync_copy(part0.at[pl.ds(s * LROWS, LROWS)],
                    table.at[pl.ds(s * LROWS, LROWS)])
    pltpu.sync_copy(iota_idx.at[s], iota_v)
    for k in range(ZCH):
        pltpu.sync_copy(part1.at[pl.ds(s * LROWS + k * CH, CH)], rows0)
        pltpu.sync_copy(rows0, table.at[iota_v.at[k]], add=True)

    @pl.when(par == 0)
    def _():
        pltpu.sync_copy(srci_e.at[wid_h], src_v)

    @pl.when(par == 1)
    def _():
        pltpu.sync_copy(srci_o.at[wid_h], src_v)

    plsc.subcore_barrier()

    @pl.when(par == 0)
    def _():
        _gather_loop(g_e, src_v, table, bufs, sems, wid_h)

    @pl.when(par == 1)
    def _():
        _gather_loop(g_o, src_v, table, bufs, sems, wid_h)


@functools.partial(
    pl.kernel,
    out_type=[jax.ShapeDtypeStruct((EP2, LAT), jnp.float32),
              jax.ShapeDtypeStruct((EP2, LAT), jnp.float32)],
    mesh=_mesh,
    scratch_types=[
        pltpu.VMEM((NCH_G, CH), jnp.int32),
        pltpu.VMEM((CH, LAT), jnp.float32),
        pltpu.VMEM((CH, LAT), jnp.float32),
        pltpu.VMEM_SHARED((NPAD, LAT), jnp.float32),
        pltpu.SemaphoreType.DMA,
        pltpu.SemaphoreType.DMA,
    ],
)
def _sc_load_gather(nlinp, srci_e, srci_o, g_e, g_o,
                    src_v, rows0, rows1, table, sem0, sem1):
    c = lax.axis_index("c")
    s = lax.axis_index("s")
    bufs = (rows0, rows1)
    sems = (sem0, sem1)
    par = s // NSH
    s_h = s - par * NSH
    wid_h = s_h * NC + c
    pltpu.sync_copy(nlinp.at[pl.ds(s * LROWS, LROWS)],
                    table.at[pl.ds(s * LROWS, LROWS)])

    @pl.when(par == 0)
    def _():
        pltpu.sync_copy(srci_e.at[wid_h], src_v)

    @pl.when(par == 1)
    def _():
        pltpu.sync_copy(srci_o.at[wid_h], src_v)

    plsc.subcore_barrier()

    @pl.when(par == 0)
    def _():
        _gather_loop(g_e, src_v, table, bufs, sems, wid_h)

    @pl.when(par == 1)
    def _():
        _gather_loop(g_o, src_v, table, bufs, sems, wid_h)


def _tc_nlin_body(x_ref, w_ref, b_ref, o_ref):
    o_ref[...] = jnp.dot(x_ref[...], w_ref[...],
                         preferred_element_type=jnp.float32) + b_ref[...]


def _tc_pre_body(ge_ref, go_ref, ef_ref, wet_ref, be_ref, wct_ref,
                 ime_ref, imo_ref, cwe_ref, cwo_ref):
    ef = ef_ref[...]
    wet = wet_ref[...]
    be = be_ref[...]
    wct = wct_ref[...]
    ime = ge_ref[...] + jnp.dot(ef[:, :16], wet,
                                preferred_element_type=jnp.float32) + be
    imo = go_ref[...] + jnp.dot(ef[:, 16:], wet,
                                preferred_element_type=jnp.float32) + be
    ime_ref[...] = ime
    imo_ref[...] = imo
    cwe_ref[...] = jnp.dot(jnp.maximum(ime, 0.0), wct,
                           preferred_element_type=jnp.float32)
    cwo_ref[...] = jnp.dot(jnp.maximum(imo, 0.0), wct,
                           preferred_element_type=jnp.float32)


def _tc_level_body(ge_ref, go_ref, cwe_ref, cwo_ref, ime_ref, imo_ref,
                   wct_ref, cb_ref, oe_ref, oo_ref, *, last):
    cb = cb_ref[...]
    he = jnp.maximum(ge_ref[...] - cwo_ref[...] + cb + ime_ref[...], 0.0)
    ho = jnp.maximum(go_ref[...] - cwe_ref[...] + cb + imo_ref[...], 0.0)
    if last:
        oe_ref[...] = he
        oo_ref[...] = ho
    else:
        wct = wct_ref[...]
        oe_ref[...] = jnp.dot(he, wct, preferred_element_type=jnp.float32)
        oo_ref[...] = jnp.dot(ho, wct, preferred_element_type=jnp.float32)


def _tc_out_body(t0_ref, t1_ref, wot_ref, ob_ref, o_ref):
    i = pl.program_id(0)
    x = jnp.maximum(t0_ref[...] + t1_ref[...], 0.0)
    h = jnp.maximum(jnp.dot(x, wot_ref[...],
                            preferred_element_type=jnp.float32) + ob_ref[...],
                    0.0)
    s = jnp.sum(h, axis=0, keepdims=True)

    @pl.when(i == 0)
    def _():
        o_ref[...] = s

    @pl.when(i > 0)
    def _():
        o_ref[...] = o_ref[...] + s

    @pl.when(i == NBLKO - 1)
    def _():
        o_ref[...] = jnp.maximum(o_ref[...], 0.0)


def _full(shape):
    return pl.BlockSpec(shape, lambda i: tuple(0 for _ in shape))


def _eblk(width=LAT):
    return pl.BlockSpec((BLK, width), lambda i: (i, 0))


def kernel(node_feat, edge_feat, edge_index, w_n2l_W, w_n2l_b, w_e2l_W,
           w_e2l_b, conv_W, conv_b, out_W, out_b):
    src = edge_index[0].astype(jnp.int32)
    dst = edge_index[1].astype(jnp.int32)
    pad_e = EP2 - E // 2
    src_e = jnp.concatenate([src[0::2], jnp.zeros((pad_e,), jnp.int32)])
    src_o = jnp.concatenate([src[1::2], jnp.zeros((pad_e,), jnp.int32)])
    dst_e = jnp.concatenate([dst[0::2], jnp.full((pad_e,), N, jnp.int32)])
    dst_o = jnp.concatenate([dst[1::2], jnp.full((pad_e,), N, jnp.int32)])
    srci_e = src_e.reshape(NWH, NCH_G, CH)
    srci_o = src_o.reshape(NWH, NCH_G, CH)
    dsti_e = dst_e.reshape(NS, NCH_S2, CH)
    dsti_o = dst_o.reshape(NS, NCH_S2, CH)
    iota_idx = jnp.arange(NPAD, dtype=jnp.int32).reshape(NS, ZCH, CH)
    zeros = jnp.zeros((CH, LAT), jnp.float32)
    ef2 = jnp.concatenate(
        [edge_feat, jnp.zeros((EP - E, 16), jnp.float32)]).reshape(EP2, 32)
    nf_p = jnp.concatenate(
        [node_feat, jnp.zeros((NPAD - N, 128), jnp.float32)])
    wnt = w_n2l_W.T
    wet = w_e2l_W.T
    wct = conv_W.T
    wot = out_W.T
    bn = w_n2l_b[None, :]
    be = w_e2l_b[None, :]
    cb = conv_b[None, :]
    ob = out_b[None, :]

    nlinp = pl.pallas_call(
        _tc_nlin_body,
        grid=(NBLKN,),
        in_specs=[pl.BlockSpec((BLKN, 128), lambda i: (i, 0)),
                  _full((128, LAT)), _full((1, LAT))],
        out_specs=pl.BlockSpec((BLKN, LAT), lambda i: (i, 0)),
        out_shape=jax.ShapeDtypeStruct((NPAD, LAT), jnp.float32),
    )(nf_p, wnt, bn)

    g0_e, g0_o = _sc_load_gather(nlinp, srci_e, srci_o)

    im_e, im_o, cw_e, cw_o = pl.pallas_call(
        _tc_pre_body,
        grid=(NBLK,),
        in_specs=[_eblk(), _eblk(), _eblk(32),
                  _full((16, LAT)), _full((1, LAT)), _full((LAT, LAT))],
        out_specs=[_eblk(), _eblk(), _eblk(), _eblk()],
        out_shape=[jax.ShapeDtypeStruct((EP2, LAT), jnp.float32)] * 4,
    )(g0_e, g0_o, ef2, wet, be, wct)

    for lv in range(3):
        p0, p1 = _sc_scatter_half(cw_e, cw_o, dsti_e, dsti_o, zeros)
        g_e, g_o = _sc_combine_gather(p0, p1, iota_idx, srci_e, srci_o)
        cw_e, cw_o = pl.pallas_call(
            functools.partial(_tc_level_body, last=(lv == 2)),
            grid=(NBLK,),
            in_specs=[_eblk(), _eblk(), _eblk(), _eblk(), _eblk(), _eblk(),
                      _full((LAT, LAT)), _full((1, LAT))],
            out_specs=[_eblk(), _eblk()],
            out_shape=[jax.ShapeDtypeStruct((EP2, LAT), jnp.float32)] * 2,
        )(g_e, g_o, cw_e, cw_o, im_e, im_o, wct, cb)

    pf0, pf1 = _sc_scatter_half(cw_e, cw_o, dsti_e, dsti_o, zeros)

    y = pl.pallas_call(
        _tc_out_body,
        grid=(NBLKO,),
        in_specs=[pl.BlockSpec((BLKO, LAT), lambda i: (i, 0)),
                  pl.BlockSpec((BLKO, LAT), lambda i: (i, 0)),
                  _full((LAT, 64)), _full((1, 64))],
        out_specs=_full((1, 64)),
        out_shape=jax.ShapeDtypeStruct((1, 64), jnp.float32),
    )(pf0, pf1, wot, ob)
    return y
